# baseline (speedup 1.0000x reference)
import jax
import jax.numpy as jnp
from jax.experimental import pallas as pl
from jax.experimental.pallas import tpu as pltpu

D_MODEL = 1024
HIDDEN = 4096
E = 8
B = 8
S = 8
T = S * B
HT = 128
N_HT = HIDDEN // HT


def _ffn_body(x_ref, kw_ref, kb_ref, vw_ref, vb_ref, o_ref, h_ref):
    t = pl.program_id(0)

    @pl.when(t == 0)
    def _init():
        vb = jnp.broadcast_to(vb_ref[...][None], (E, B, D_MODEL))
        o_ref[...] = jnp.broadcast_to(
            vb.reshape(1, T, D_MODEL), (S, T, D_MODEL))

    kwt = jnp.swapaxes(kw_ref[...].reshape(D_MODEL, E, HT), 0, 1)
    for e in range(E):
        h = jnp.dot(x_ref[pl.ds(e * T, T), :], kwt[e],
                    preferred_element_type=jnp.float32)
        h_ref[:, pl.ds(e * B, B), :] = h.reshape(S, B, HT)

    vwt = jnp.swapaxes(vw_ref[...].reshape(HT, S, D_MODEL), 0, 1)
    for s in range(S):
        g = jax.nn.gelu(h_ref[s] + kb_ref[s][None])
        o_ref[pl.ds(s, 1), :, :] += jnp.dot(
            g, vwt[s], preferred_element_type=jnp.float32
        )[None]


def kernel(x, keys_w, key_bias, values_w, value_bias):
    xe = jnp.transpose(x, (2, 1, 0, 3)).reshape(E * T, D_MODEL)
    kw2 = keys_w.reshape(D_MODEL * E, HIDDEN)
    vw2 = values_w.reshape(HIDDEN * S, D_MODEL)

    out = pl.pallas_call(
        _ffn_body,
        grid=(N_HT,),
        in_specs=[
            pl.BlockSpec((E * T, D_MODEL), lambda t: (0, 0)),
            pl.BlockSpec((D_MODEL * E, HT), lambda t: (0, t)),
            pl.BlockSpec((S, HT), lambda t: (0, t)),
            pl.BlockSpec((HT * S, D_MODEL), lambda t: (t, 0)),
            pl.BlockSpec((B, D_MODEL), lambda t: (0, 0)),
        ],
        out_specs=pl.BlockSpec((S, T, D_MODEL), lambda t: (0, 0, 0)),
        out_shape=jax.ShapeDtypeStruct((S, T, D_MODEL), jnp.float32),
        scratch_shapes=[pltpu.VMEM((S, T, HT), jnp.float32)],
    )(xe, kw2, key_bias, vw2, value_bias)

    return out.reshape(S, E, B, D_MODEL)


# speedup vs baseline: 1.1122x; 1.1122x over previous
import jax
import jax.numpy as jnp
from jax.experimental import pallas as pl
from jax.experimental.pallas import tpu as pltpu

D_MODEL = 1024
HIDDEN = 4096
E = 8
B = 8
S = 8
T = S * B
HT = 256
N_HT = HIDDEN // HT


def _ffn_body(x_ref, kw_ref, kb_ref, vw_ref, vb_ref, o_ref, h_ref):
    t = pl.program_id(0)

    @pl.when(t == 0)
    def _init():
        vb = jnp.broadcast_to(vb_ref[...][None], (E, B, D_MODEL))
        o_ref[...] = jnp.broadcast_to(
            vb.reshape(1, T, D_MODEL), (S, T, D_MODEL))

    kwt = jnp.swapaxes(kw_ref[...].reshape(D_MODEL, E, HT), 0, 1)
    for e in range(E):
        h = jnp.dot(x_ref[pl.ds(e * T, T), :], kwt[e],
                    preferred_element_type=jnp.float32)
        h_ref[:, pl.ds(e * B, B), :] = h.reshape(S, B, HT)

    vwt = jnp.swapaxes(vw_ref[...].reshape(HT, S, D_MODEL), 0, 1)
    for s in range(S):
        g = jax.nn.gelu(h_ref[s] + kb_ref[s][None])
        o_ref[pl.ds(s, 1), :, :] += jnp.dot(
            g, vwt[s], preferred_element_type=jnp.float32
        )[None]


def kernel(x, keys_w, key_bias, values_w, value_bias):
    xe = jnp.transpose(x, (2, 1, 0, 3)).reshape(E * T, D_MODEL)
    kw2 = keys_w.reshape(D_MODEL * E, HIDDEN)
    vw2 = values_w.reshape(HIDDEN * S, D_MODEL)

    out = pl.pallas_call(
        _ffn_body,
        grid=(N_HT,),
        in_specs=[
            pl.BlockSpec((E * T, D_MODEL), lambda t: (0, 0)),
            pl.BlockSpec((D_MODEL * E, HT), lambda t: (0, t)),
            pl.BlockSpec((S, HT), lambda t: (0, t)),
            pl.BlockSpec((HT * S, D_MODEL), lambda t: (t, 0)),
            pl.BlockSpec((B, D_MODEL), lambda t: (0, 0)),
        ],
        out_specs=pl.BlockSpec((S, T, D_MODEL), lambda t: (0, 0, 0)),
        out_shape=jax.ShapeDtypeStruct((S, T, D_MODEL), jnp.float32),
        scratch_shapes=[pltpu.VMEM((S, T, HT), jnp.float32)],
    )(xe, kw2, key_bias, vw2, value_bias)

    return out.reshape(S, E, B, D_MODEL)


# speedup vs baseline: 1.1510x; 1.0349x over previous
import jax
import jax.numpy as jnp
from jax.experimental import pallas as pl
from jax.experimental.pallas import tpu as pltpu

D_MODEL = 1024
HIDDEN = 4096
E = 8
B = 8
S = 8
T = S * B
HT = 256
N_HT = HIDDEN // HT


def _ffn_body(x_ref, kw_ref, kb_ref, vw_ref, vb_ref, o_ref, h_ref, xs_ref):
    t = pl.program_id(0)

    @pl.when(t == 0)
    def _init():
        vb = jnp.broadcast_to(vb_ref[...][None], (E, B, D_MODEL))
        o_ref[...] = jnp.broadcast_to(
            vb.reshape(1, T, D_MODEL), (S, T, D_MODEL))
        xt = jnp.swapaxes(x_ref[...].reshape(B * S, E, D_MODEL), 0, 1)
        xs = jnp.swapaxes(xt.reshape(E, B, S, D_MODEL), 1, 2)
        xs_ref[...] = xs.reshape(E * T, D_MODEL)

    kwt = jnp.swapaxes(kw_ref[...].reshape(D_MODEL, E, HT), 0, 1)
    for e in range(E):
        h = jnp.dot(xs_ref[pl.ds(e * T, T), :], kwt[e],
                    preferred_element_type=jnp.float32)
        h_ref[:, pl.ds(e * B, B), :] = h.reshape(S, B, HT)

    vwt = jnp.swapaxes(vw_ref[...].reshape(HT, S, D_MODEL), 0, 1)
    for s in range(S):
        g = jax.nn.gelu(h_ref[s] + kb_ref[s][None])
        o_ref[pl.ds(s, 1), :, :] += jnp.dot(
            g, vwt[s], preferred_element_type=jnp.float32
        )[None]


def kernel(x, keys_w, key_bias, values_w, value_bias):
    x2 = x.reshape(B * S * E, D_MODEL)
    kw2 = keys_w.reshape(D_MODEL * E, HIDDEN)
    vw2 = values_w.reshape(HIDDEN * S, D_MODEL)

    out = pl.pallas_call(
        _ffn_body,
        grid=(N_HT,),
        in_specs=[
            pl.BlockSpec((B * S * E, D_MODEL), lambda t: (0, 0)),
            pl.BlockSpec((D_MODEL * E, HT), lambda t: (0, t)),
            pl.BlockSpec((S, HT), lambda t: (0, t)),
            pl.BlockSpec((HT * S, D_MODEL), lambda t: (t, 0)),
            pl.BlockSpec((B, D_MODEL), lambda t: (0, 0)),
        ],
        out_specs=pl.BlockSpec((S, T, D_MODEL), lambda t: (0, 0, 0)),
        out_shape=jax.ShapeDtypeStruct((S, T, D_MODEL), jnp.float32),
        scratch_shapes=[
            pltpu.VMEM((S, T, HT), jnp.float32),
            pltpu.VMEM((E * T, D_MODEL), jnp.float32),
        ],
    )(x2, kw2, key_bias, vw2, value_bias)

    return out.reshape(S, E, B, D_MODEL)
